# baseline (speedup 1.0000x reference)
import functools

import jax
import jax.numpy as jnp
from jax import lax
from jax.experimental import pallas as pl
from jax.experimental.pallas import tpu as pltpu
from jax.experimental.pallas import tpu_sc as plsc

VOCAB1 = 100001
D = 64
DP = 128
B, L = 4096, 50
LP = 56
NP = B * LP
NC, NS = 2, 16
NW = NC * NS
PER_W = NP // NW
IR_PER_W = PER_W // DP
CH = 256
GPC = CH // DP
NCH = PER_W // CH


def _gather_grid(table_hbm, idx_hbm, out_hbm, idx_v, rows_v, g0, g1, w0, w1):
    wid = lax.axis_index("s") * NC + lax.axis_index("c")
    base = wid * PER_W
    irbase = wid * IR_PER_W
    gsem = (g0, g1)
    wsem = (w0, w1)

    pltpu.sync_copy(idx_hbm.at[pl.ds(irbase, IR_PER_W)], idx_v)

    def start_gathers(j, b):
        return [
            pltpu.async_copy(
                table_hbm.at[idx_v.at[j * GPC + k]],
                rows_v.at[b].at[pl.ds(k * DP, DP)],
                gsem[b],
            )
            for k in range(GPC)
        ]

    def start_writeback(j, b):
        return pltpu.async_copy(
            rows_v.at[b],
            out_hbm.at[pl.ds(base + j * CH, CH)],
            wsem[b],
        )

    gh = [None] * NCH
    wh = [None] * NCH
    gh[0] = start_gathers(0, 0)
    for j in range(NCH):
        b = j % 2
        for h in gh[j]:
            h.wait()
        wh[j] = start_writeback(j, b)
        if j + 1 < NCH:
            if j >= 1:
                wh[j - 1].wait()
            gh[j + 1] = start_gathers(j + 1, 1 - b)
    wh[NCH - 2].wait()
    wh[NCH - 1].wait()


def kernel(indices, vectors):
    table = jnp.pad(vectors, ((0, 0), (0, DP - D)))
    idx = jnp.pad(indices.astype(jnp.int32), ((0, 0), (0, LP - L)))
    idx = idx.reshape(NP // DP, DP)
    mesh = plsc.VectorSubcoreMesh(core_axis_name="c", subcore_axis_name="s")
    run = functools.partial(
        pl.kernel,
        mesh=mesh,
        out_type=jax.ShapeDtypeStruct((NP, DP), jnp.float32),
        scratch_types=[
            pltpu.VMEM((IR_PER_W, DP), jnp.int32),
            pltpu.VMEM((2, CH, DP), jnp.float32),
            pltpu.SemaphoreType.DMA,
            pltpu.SemaphoreType.DMA,
            pltpu.SemaphoreType.DMA,
            pltpu.SemaphoreType.DMA,
        ],
    )(_gather_grid)
    out = run(table, idx)
    return out.reshape(B, LP, DP)[:, :L, :D]
